# Initial kernel scaffold; baseline (speedup 1.0000x reference)
import jax
import jax.numpy as jnp
from jax.experimental import pallas as pl


def kernel(x, pe):
    raise NotImplementedError("write your pallas kernel here")


# speedup vs baseline: 1.4911x; 1.4911x over previous
import jax
import jax.numpy as jnp
from jax.experimental import pallas as pl

BLK = 512


def _add_kernel(x_ref, pe_ref, o_ref):
    o_ref[...] = x_ref[...] + pe_ref[...]


def kernel(x, pe):
    B, S, H = x.shape
    grid = (S // BLK, B)
    return pl.pallas_call(
        _add_kernel,
        grid=grid,
        in_specs=[
            pl.BlockSpec((1, BLK, H), lambda j, b: (b, j, 0)),
            pl.BlockSpec((BLK, H), lambda j, b: (j, 0)),
        ],
        out_specs=pl.BlockSpec((1, BLK, H), lambda j, b: (b, j, 0)),
        out_shape=jax.ShapeDtypeStruct((B, S, H), x.dtype),
    )(x, pe)


# speedup vs baseline: 1.6640x; 1.1159x over previous
import jax
import jax.numpy as jnp
from jax.experimental import pallas as pl

BLK = 1024


def _add_kernel(x_ref, pe_ref, o_ref):
    o_ref[...] = x_ref[...] + pe_ref[...]


def kernel(x, pe):
    B, S, H = x.shape
    grid = (S // BLK, B)
    return pl.pallas_call(
        _add_kernel,
        grid=grid,
        in_specs=[
            pl.BlockSpec((1, BLK, H), lambda j, b: (b, j, 0)),
            pl.BlockSpec((BLK, H), lambda j, b: (j, 0)),
        ],
        out_specs=pl.BlockSpec((1, BLK, H), lambda j, b: (b, j, 0)),
        out_shape=jax.ShapeDtypeStruct((B, S, H), x.dtype),
    )(x, pe)


# speedup vs baseline: 1.7366x; 1.0436x over previous
import jax
import jax.numpy as jnp
from jax.experimental import pallas as pl

BLK = 2048


def _add_kernel(x_ref, pe_ref, o_ref):
    o_ref[...] = x_ref[...] + pe_ref[...]


def kernel(x, pe):
    B, S, H = x.shape
    grid = (S // BLK, B)
    return pl.pallas_call(
        _add_kernel,
        grid=grid,
        in_specs=[
            pl.BlockSpec((1, BLK, H), lambda j, b: (b, j, 0)),
            pl.BlockSpec((BLK, H), lambda j, b: (j, 0)),
        ],
        out_specs=pl.BlockSpec((1, BLK, H), lambda j, b: (b, j, 0)),
        out_shape=jax.ShapeDtypeStruct((B, S, H), x.dtype),
    )(x, pe)
